# baseline (speedup 1.0000x reference)
import jax
import jax.numpy as jnp
from jax.experimental import pallas as pl
from jax.experimental.pallas import tpu as pltpu

_N, _NFEAT, _NHID, _NCLASS = 4096, 128, 128, 64
_R = 512
_NBLK = _N // _R
_TOT = 3 * _NBLK
_K = 6


def _ln_relu(y, g, b):
    mu = jnp.mean(y, axis=-1, keepdims=True)
    d = y - mu
    var = jnp.mean(d * d, axis=-1, keepdims=True)
    return jnp.maximum(d * jax.lax.rsqrt(var + 1e-5) * g + b, 0.0)


def _gcn_body(adj_ref, x_ref, w1_ref, wnext_ref, bias_ref, lng_ref, lnb_ref,
              wo_ref, bo_ref, out_ref, z_a, z_b, bufs, sems):
    l = pl.program_id(0)
    b = pl.program_id(1)
    i = l * _NBLK + b

    def start_copy(lj, bj, slot):
        pltpu.make_async_copy(
            adj_ref.at[lj, pl.ds(bj * _R, _R), :],
            bufs.at[slot],
            sems.at[slot],
        ).start()

    @pl.when(i == 0)
    def _():
        for j in range(_K - 1):
            start_copy(j // _NBLK, j % _NBLK, j)

    j = i + _K - 1

    @pl.when(j < _TOT)
    def _():
        start_copy(j // _NBLK, j % _NBLK, j % _K)

    @pl.when(i == 0)
    def _():
        z_a[...] = jnp.dot(x_ref[...], w1_ref[...],
                           preferred_element_type=jnp.float32)

    slot = i % _K
    pltpu.make_async_copy(
        adj_ref.at[l, pl.ds(b * _R, _R), :],
        bufs.at[slot],
        sems.at[slot],
    ).wait()

    g = lng_ref[0, :]
    beta = lnb_ref[0, :]

    def layer_h(z_ref):
        y = jnp.dot(bufs[slot], z_ref[...],
                    preferred_element_type=jnp.float32)
        return _ln_relu(y + bias_ref[0, 0, :], g, beta)

    @pl.when(l == 0)
    def _():
        h = layer_h(z_a)
        z_b[pl.ds(b * _R, _R), :] = jnp.dot(
            h, wnext_ref[0], preferred_element_type=jnp.float32)

    @pl.when(l == 1)
    def _():
        h = layer_h(z_b)
        z_a[pl.ds(b * _R, _R), :] = jnp.dot(
            h, wnext_ref[0], preferred_element_type=jnp.float32)

    @pl.when(l == 2)
    def _():
        h = layer_h(z_a)
        logits = jnp.dot(h, wo_ref[...],
                         preferred_element_type=jnp.float32) + bo_ref[0, :]
        m = jnp.max(logits, axis=-1, keepdims=True)
        e = jnp.exp(logits - m)
        t = jnp.sum(e, axis=-1, keepdims=True)
        out_ref[...] = logits - m - jnp.log(t)


def kernel(x, adj, W1, b1, W2, b2, W3, b3, ln_g, ln_b, Wo, bo):
    wnext = jnp.stack([W2, W3])
    bias = jnp.stack([b1, b2, b3])[:, None, :]
    lng = ln_g.reshape(1, _NHID)
    lnb = ln_b.reshape(1, _NHID)
    bo2 = bo.reshape(1, _NCLASS)

    return pl.pallas_call(
        _gcn_body,
        grid=(3, _NBLK),
        in_specs=[
            pl.BlockSpec(memory_space=pl.ANY),
            pl.BlockSpec((_N, _NFEAT), lambda l, b: (0, 0)),
            pl.BlockSpec((_NFEAT, _NHID), lambda l, b: (0, 0)),
            pl.BlockSpec((1, _NHID, _NHID),
                         lambda l, b: (jnp.minimum(l, 1), 0, 0)),
            pl.BlockSpec((1, 1, _NHID), lambda l, b: (l, 0, 0)),
            pl.BlockSpec((1, _NHID), lambda l, b: (0, 0)),
            pl.BlockSpec((1, _NHID), lambda l, b: (0, 0)),
            pl.BlockSpec((_NHID, _NCLASS), lambda l, b: (0, 0)),
            pl.BlockSpec((1, _NCLASS), lambda l, b: (0, 0)),
        ],
        out_specs=pl.BlockSpec((_R, _NCLASS), lambda l, b: (b, 0)),
        out_shape=jax.ShapeDtypeStruct((_N, _NCLASS), jnp.float32),
        scratch_shapes=[
            pltpu.VMEM((_N, _NHID), jnp.float32),
            pltpu.VMEM((_N, _NHID), jnp.float32),
            pltpu.VMEM((_K, _R, _N), jnp.float32),
            pltpu.SemaphoreType.DMA((_K,)),
        ],
        compiler_params=pltpu.CompilerParams(
            dimension_semantics=("arbitrary", "arbitrary")),
    )(adj, x, W1, wnext, bias, lng, lnb, Wo, bo2)


# speedup vs baseline: 1.0625x; 1.0625x over previous
import jax
import jax.numpy as jnp
from jax.experimental import pallas as pl
from jax.experimental.pallas import tpu as pltpu

_N, _NFEAT, _NHID, _NCLASS = 4096, 128, 128, 64
_R = 1024
_NWAY = 1
_RB = _R * _NWAY
_NBLK = _N // _RB


def _ln_relu(y, g, b):
    mu = jnp.mean(y, axis=-1, keepdims=True)
    d = y - mu
    var = jnp.mean(d * d, axis=-1, keepdims=True)
    return jnp.maximum(d * jax.lax.rsqrt(var + 1e-5) * g + b, 0.0)


def _gcn_body(*refs):
    adj_refs = refs[:_NWAY]
    (x_ref, w1_ref, wnext_ref, bias_ref, lng_ref, lnb_ref, wo_ref, bo_ref,
     out_ref, z_a, z_b) = refs[_NWAY:]
    l = pl.program_id(0)
    b = pl.program_id(1)

    @pl.when((l == 0) & (b == 0))
    def _():
        z_a[...] = jnp.dot(x_ref[...], w1_ref[...],
                           preferred_element_type=jnp.float32)

    g = lng_ref[0, :]
    beta = lnb_ref[0, :]

    def layer_h(z_ref, s):
        y = jnp.dot(adj_refs[s][0], z_ref[...],
                    preferred_element_type=jnp.float32)
        return _ln_relu(y + bias_ref[0, 0, :], g, beta)

    @pl.when(l == 0)
    def _():
        for s in range(_NWAY):
            h = layer_h(z_a, s)
            z_b[pl.ds(b * _RB + s * _R, _R), :] = jnp.dot(
                h, wnext_ref[0], preferred_element_type=jnp.float32)

    @pl.when(l == 1)
    def _():
        for s in range(_NWAY):
            h = layer_h(z_b, s)
            z_a[pl.ds(b * _RB + s * _R, _R), :] = jnp.dot(
                h, wnext_ref[0], preferred_element_type=jnp.float32)

    @pl.when(l == 2)
    def _():
        for s in range(_NWAY):
            h = layer_h(z_a, s)
            logits = jnp.dot(h, wo_ref[...],
                             preferred_element_type=jnp.float32) + bo_ref[0, :]
            m = jnp.max(logits, axis=-1, keepdims=True)
            e = jnp.exp(logits - m)
            t = jnp.sum(e, axis=-1, keepdims=True)
            out_ref[pl.ds(s * _R, _R), :] = logits - m - jnp.log(t)


def _adj_spec(s):
    return pl.BlockSpec((1, _R, _N), lambda l, b, _s=s: (l, b * _NWAY + _s, 0))


def kernel(x, adj, W1, b1, W2, b2, W3, b3, ln_g, ln_b, Wo, bo):
    wnext = jnp.stack([W2, W3])
    bias = jnp.stack([b1, b2, b3])[:, None, :]
    lng = ln_g.reshape(1, _NHID)
    lnb = ln_b.reshape(1, _NHID)
    bo2 = bo.reshape(1, _NCLASS)

    return pl.pallas_call(
        _gcn_body,
        grid=(3, _NBLK),
        in_specs=[_adj_spec(s) for s in range(_NWAY)] + [
            pl.BlockSpec((_N, _NFEAT), lambda l, b: (0, 0)),
            pl.BlockSpec((_NFEAT, _NHID), lambda l, b: (0, 0)),
            pl.BlockSpec((1, _NHID, _NHID),
                         lambda l, b: (jnp.minimum(l, 1), 0, 0)),
            pl.BlockSpec((1, 1, _NHID), lambda l, b: (l, 0, 0)),
            pl.BlockSpec((1, _NHID), lambda l, b: (0, 0)),
            pl.BlockSpec((1, _NHID), lambda l, b: (0, 0)),
            pl.BlockSpec((_NHID, _NCLASS), lambda l, b: (0, 0)),
            pl.BlockSpec((1, _NCLASS), lambda l, b: (0, 0)),
        ],
        out_specs=pl.BlockSpec((_RB, _NCLASS), lambda l, b: (b, 0)),
        out_shape=jax.ShapeDtypeStruct((_N, _NCLASS), jnp.float32),
        scratch_shapes=[
            pltpu.VMEM((_N, _NHID), jnp.float32),
            pltpu.VMEM((_N, _NHID), jnp.float32),
        ],
        compiler_params=pltpu.CompilerParams(
            dimension_semantics=("arbitrary", "arbitrary")),
    )(*([adj] * _NWAY), x, W1, wnext, bias, lng, lnb, Wo, bo2)
